# Initial kernel scaffold; baseline (speedup 1.0000x reference)
import jax
import jax.numpy as jnp
from jax.experimental import pallas as pl


def kernel(embeddings, codebooks):
    raise NotImplementedError("write your pallas kernel here")


# speedup vs baseline: 2.2582x; 2.2582x over previous
import jax
import jax.numpy as jnp
from jax.experimental import pallas as pl

_N_CB = 4
_K = 1024
_E = 32


def _rvq_kernel(x_ref, cb_ref, out_ref):
    xb = x_ref[0]
    res = xb.T
    quant = jnp.zeros_like(res)
    tok = res.shape[0]
    for i in range(_N_CB):
        cb = cb_ref[i]
        a2 = jnp.sum(res * res, axis=1, keepdims=True)
        b2 = jnp.sum(cb * cb, axis=1)[None, :]
        s = jnp.dot(res, cb.T, preferred_element_type=jnp.float32)
        d = a2 - 2.0 * s + b2
        m = jnp.min(d, axis=1, keepdims=True)
        iota = jax.lax.broadcasted_iota(jnp.int32, (tok, _K), 1)
        idx = jnp.min(jnp.where(d == m, iota, _K), axis=1)
        onehot = (iota == idx[:, None]).astype(jnp.float32)
        q = jnp.dot(onehot, cb, preferred_element_type=jnp.float32)
        res = res - q
        quant = quant + q
    out_ref[0] = quant.T


def kernel(embeddings, codebooks):
    B, E, H, W = embeddings.shape
    T = H * W
    x = embeddings.reshape(B, E, T)
    out = pl.pallas_call(
        _rvq_kernel,
        grid=(B,),
        in_specs=[
            pl.BlockSpec((1, E, T), lambda b: (b, 0, 0)),
            pl.BlockSpec((_N_CB, _K, _E), lambda b: (0, 0, 0)),
        ],
        out_specs=pl.BlockSpec((1, E, T), lambda b: (b, 0, 0)),
        out_shape=jax.ShapeDtypeStruct((B, E, T), jnp.float32),
    )(x, codebooks)
    return out.reshape(B, E, H, W)
